# baseline (speedup 1.0000x reference)
import jax
import jax.numpy as jnp
from jax import lax
from jax.experimental import pallas as pl
from jax.experimental.pallas import tpu as pltpu
from jax.experimental.pallas import tpu_sc as plsc

B, N, C = 4, 16384, 256
NE = 64
K = 32
NBLK = 2048
NJ = N // NBLK
SUB = 128
LANE = 128
BIG = 1 << 30
NEG = float("-inf")
NROWS = B * NE
NWORK = 32
RPW = NROWS // NWORK


def _bf16_round(v):
    u = lax.bitcast_convert_type(v, jnp.int32)
    r = (u + 0x7FFF + ((u >> 16) & 1)) & ~0xFFFF
    return lax.bitcast_convert_type(r.astype(jnp.int32), jnp.float32)

def _ka1(x_ref, ct_ref, inv_ref, s_ref):
    x2 = x_ref[0]
    ct = ct_ref[0]
    ss = jnp.sum(x2 * x2, axis=1, keepdims=True)
    nrm = jnp.maximum(jnp.sqrt(ss), 1e-12)
    inv_ref[0] = nrm
    ctn = ct / jnp.maximum(jnp.sqrt(jnp.sum(ct * ct)), 1e-12)
    xn = x2 / nrm
    s_ref[0] = lax.dot_general(
        _bf16_round(xn), _bf16_round(ctn),
        (((1,), (1,)), ((), ())),
        preferred_element_type=jnp.float32)


def _ka2(s_ref, cidx_ref, cflat_ref):
    b = pl.program_id(0)
    s2 = s_ref[0]
    r = lax.broadcasted_iota(jnp.int32, (SUB, LANE), 0)
    c = lax.broadcasted_iota(jnp.int32, (SUB, LANE), 1)
    n2d = r * LANE + c

    def body(i, s):
        m = jnp.max(s)
        sel = jnp.min(jnp.where(s == m, n2d, BIG))
        cidx_ref[0, 0, i] = sel
        cflat_ref[0, 0, i] = sel + b * N
        return jnp.where(n2d == sel, NEG, s)

    lax.fori_loop(0, NE, body, s2)


def _kg_sc(x2_ref, cflat_ref, cout_ref, idxv, rowsv, sem):
    wid = lax.axis_index("s") * 2 + lax.axis_index("c")
    base = wid * (NROWS // NWORK)
    pltpu.sync_copy(cflat_ref.at[pl.ds(base, RPW)], idxv)
    pltpu.async_copy(x2_ref.at[idxv], rowsv, sem).wait()
    pltpu.sync_copy(rowsv, cout_ref.at[pl.ds(base, RPW)])


def _kb(cidx_ref, cr_ref, inv_ref, x_ref, temp_ref, out_ref):
    j = pl.program_id(1)
    cr = cr_ref[0]
    ssc = jnp.sum(cr * cr, axis=1, keepdims=True)
    cnrm = jnp.maximum(jnp.sqrt(ssc), 1e-12)
    xb = x_ref[0]
    xnrm = inv_ref[0]
    cn = cr / cnrm
    fn = xb / xnrm
    dot = lax.dot_general(
        _bf16_round(cn), _bf16_round(fn),
        (((1,), (1,)), ((), ())),
        preferred_element_type=jnp.float32)
    sem = dot / temp_ref[0]
    nrow = j * NBLK + lax.broadcasted_iota(jnp.int32, (1, NBLK), 1)
    zn = 2.0 * (nrow >> 10).astype(jnp.float32)
    yn = ((nrow >> 5) & 31).astype(jnp.float32)
    xn = (nrow & 31).astype(jnp.float32)
    ci = cidx_ref[0, 0][:, None]
    zc = 2.0 * (ci >> 10).astype(jnp.float32)
    yc = ((ci >> 5) & 31).astype(jnp.float32)
    xc = (ci & 31).astype(jnp.float32)
    dz = zn - zc
    dy = yn - yc
    dx = xn - xc
    dist = jnp.sqrt(dz * dz + dy * dy + dx * dx)
    dzm = jnp.maximum(zc, 30.0 - zc)
    dym = jnp.maximum(yc, 31.0 - yc)
    dxm = jnp.maximum(xc, 31.0 - xc)
    maxd = jnp.sqrt(dzm * dzm + dym * dym + dxm * dxm)
    sd = dist / (maxd + 1e-8)
    out_ref[0] = 0.9 * sem + 0.1 * (1.0 - sd)


def _kt(ct_ref, v_ref, n_ref):
    nco = lax.broadcasted_iota(jnp.int32, (NE, N), 1)

    def body(i, carry):
        mp, np_ = carry
        s = ct_ref[0]
        sm = jnp.where((s < mp) | ((s == mp) & (nco > np_)), s, NEG)
        m = jnp.max(sm, axis=1, keepdims=True)
        sel = jnp.min(jnp.where(sm == m, nco, BIG), axis=1, keepdims=True)
        return m, sel

    m, sel = lax.fori_loop(0, K, body,
                           (jnp.full((NE, 1), jnp.inf, jnp.float32),
                            jnp.full((NE, 1), -1, jnp.int32)))
    v_ref[0] = m.reshape(1, NE)
    n_ref[0] = sel.reshape(1, NE)


def _kc(ct_ref, v_ref, n_ref, h_ref):
    j = pl.program_id(1)
    cb = ct_ref[0]
    v = v_ref[0, 0][:, None]
    nn = n_ref[0, 0][:, None]
    nrow = j * NBLK + lax.broadcasted_iota(jnp.int32, (1, NBLK), 1)
    keep = (cb > v) | ((cb == v) & (nrow <= nn))
    h_ref[0] = keep.astype(jnp.float32).T


def kernel(node_features, class_token, temperature):
    inv3, sarr = pl.pallas_call(
        _ka1,
        grid=(B, NJ),
        in_specs=[
            pl.BlockSpec((1, NBLK, C), lambda b, j: (b, j, 0)),
            pl.BlockSpec((1, 1, C), lambda b, j: (0, 0, 0)),
        ],
        out_specs=[
            pl.BlockSpec((1, NBLK, 1), lambda b, j: (b, j, 0)),
            pl.BlockSpec((1, NBLK, 1), lambda b, j: (b, j, 0)),
        ],
        out_shape=[
            jax.ShapeDtypeStruct((B, N, 1), jnp.float32),
            jax.ShapeDtypeStruct((B, N, 1), jnp.float32),
        ],
    )(node_features, class_token)

    cidx, cflat = pl.pallas_call(
        _ka2,
        grid=(B,),
        in_specs=[pl.BlockSpec((1, SUB, LANE), lambda b: (b, 0, 0))],
        out_specs=[
            pl.BlockSpec((1, 1, NE), lambda b: (b, 0, 0),
                         memory_space=pltpu.SMEM),
            pl.BlockSpec((1, 1, NE), lambda b: (b, 0, 0),
                         memory_space=pltpu.SMEM),
        ],
        out_shape=[
            jax.ShapeDtypeStruct((B, 1, NE), jnp.int32),
            jax.ShapeDtypeStruct((B, 1, NE), jnp.int32),
        ],
    )(sarr.reshape(B, SUB, LANE))

    mesh = plsc.VectorSubcoreMesh(core_axis_name="c", subcore_axis_name="s")
    centers_flat = pl.kernel(
        _kg_sc,
        mesh=mesh,
        out_type=jax.ShapeDtypeStruct((NROWS, C), jnp.float32),
        scratch_types=[
            pltpu.VMEM((RPW,), jnp.int32),
            pltpu.VMEM((RPW, C), jnp.float32),
            pltpu.SemaphoreType.DMA,
        ],
    )(node_features.reshape(B * N, C), cflat.reshape(NROWS))
    centers = centers_flat.reshape(B, NE, C)

    comb = pl.pallas_call(
        _kb,
        grid=(B, NJ),
        in_specs=[
            pl.BlockSpec((1, 1, NE), lambda b, j: (b, 0, 0)),
            pl.BlockSpec((1, NE, C), lambda b, j: (b, 0, 0)),
            pl.BlockSpec((1, NBLK, 1), lambda b, j: (b, j, 0)),
            pl.BlockSpec((1, NBLK, C), lambda b, j: (b, j, 0)),
            pl.BlockSpec(memory_space=pltpu.SMEM),
        ],
        out_specs=pl.BlockSpec((1, NE, NBLK), lambda b, j: (b, 0, j)),
        out_shape=jax.ShapeDtypeStruct((B, NE, N), jnp.float32),
    )(cidx, centers, inv3, node_features, temperature)

    v32, n32 = pl.pallas_call(
        _kt,
        grid=(B,),
        in_specs=[pl.BlockSpec((1, NE, N), lambda b: (b, 0, 0))],
        out_specs=[
            pl.BlockSpec((1, 1, NE), lambda b: (b, 0, 0)),
            pl.BlockSpec((1, 1, NE), lambda b: (b, 0, 0)),
        ],
        out_shape=[
            jax.ShapeDtypeStruct((B, 1, NE), jnp.float32),
            jax.ShapeDtypeStruct((B, 1, NE), jnp.int32),
        ],
    )(comb)

    H = pl.pallas_call(
        _kc,
        grid=(B, NJ),
        in_specs=[
            pl.BlockSpec((1, NE, NBLK), lambda b, j: (b, 0, j)),
            pl.BlockSpec((1, 1, NE), lambda b, j: (b, 0, 0)),
            pl.BlockSpec((1, 1, NE), lambda b, j: (b, 0, 0)),
        ],
        out_specs=pl.BlockSpec((1, NBLK, NE), lambda b, j: (b, j, 0)),
        out_shape=jax.ShapeDtypeStruct((B, N, NE), jnp.float32),
    )(comb, v32, n32)
    return H


# speedup vs baseline: 1.1318x; 1.1318x over previous
import jax
import jax.numpy as jnp
from jax import lax
from jax.experimental import pallas as pl
from jax.experimental.pallas import tpu as pltpu
from jax.experimental.pallas import tpu_sc as plsc

B, N, C = 4, 16384, 256
NE = 64
K = 32
NBLK = 2048
NJ = N // NBLK
SUB = 128
LANE = 128
BIG = 1 << 30
NEG = float("-inf")
NROWS = B * NE
NWORK = 32
RPW = NROWS // NWORK


def _bf16_round(v):
    u = lax.bitcast_convert_type(v, jnp.int32)
    r = (u + 0x7FFF + ((u >> 16) & 1)) & ~0xFFFF
    return lax.bitcast_convert_type(r.astype(jnp.int32), jnp.float32)

def _ka1(x_ref, ct_ref, inv_ref, s_ref):
    x2 = x_ref[0]
    ct = ct_ref[0]
    ss = jnp.sum(x2 * x2, axis=1, keepdims=True)
    nrm = jnp.maximum(jnp.sqrt(ss), 1e-12)
    inv_ref[0] = nrm
    ctn = ct / jnp.maximum(jnp.sqrt(jnp.sum(ct * ct)), 1e-12)
    xn = x2 / nrm
    s_ref[0] = lax.dot_general(
        _bf16_round(xn), _bf16_round(ctn),
        (((1,), (1,)), ((), ())),
        preferred_element_type=jnp.float32)


def _ka2(s_ref, cidx_ref, cflat_ref):
    b = pl.program_id(0)
    s2 = s_ref[0]
    r = lax.broadcasted_iota(jnp.int32, (SUB, LANE), 0)
    c = lax.broadcasted_iota(jnp.int32, (SUB, LANE), 1)
    n2d = r * LANE + c

    def body(i, s):
        m = jnp.max(s)
        sel = jnp.min(jnp.where(s == m, n2d, BIG))
        cidx_ref[0, 0, i] = sel
        cflat_ref[0, 0, i] = sel + b * N
        return jnp.where(n2d == sel, NEG, s)

    lax.fori_loop(0, NE, body, s2)


def _kg_sc(x2_ref, cflat_ref, cout_ref, idxv, rowsv, sem):
    wid = lax.axis_index("s") * 2 + lax.axis_index("c")
    base = wid * (NROWS // NWORK)
    pltpu.sync_copy(cflat_ref.at[pl.ds(base, RPW)], idxv)
    pltpu.async_copy(x2_ref.at[idxv], rowsv, sem).wait()
    pltpu.sync_copy(rowsv, cout_ref.at[pl.ds(base, RPW)])


def _kb(cidx_ref, cr_ref, inv_ref, x_ref, temp_ref, out_ref):
    j = pl.program_id(1)
    cr = cr_ref[0]
    ssc = jnp.sum(cr * cr, axis=1, keepdims=True)
    cnrm = jnp.maximum(jnp.sqrt(ssc), 1e-12)
    xb = x_ref[0]
    xnrm = inv_ref[0]
    cn = cr / cnrm
    fn = xb / xnrm
    dot = lax.dot_general(
        _bf16_round(cn), _bf16_round(fn),
        (((1,), (1,)), ((), ())),
        preferred_element_type=jnp.float32)
    sem = dot / temp_ref[0]
    nrow = j * NBLK + lax.broadcasted_iota(jnp.int32, (1, NBLK), 1)
    zn = 2.0 * (nrow >> 10).astype(jnp.float32)
    yn = ((nrow >> 5) & 31).astype(jnp.float32)
    xn = (nrow & 31).astype(jnp.float32)
    ci = cidx_ref[0, 0][:, None]
    zc = 2.0 * (ci >> 10).astype(jnp.float32)
    yc = ((ci >> 5) & 31).astype(jnp.float32)
    xc = (ci & 31).astype(jnp.float32)
    dz = zn - zc
    dy = yn - yc
    dx = xn - xc
    dist = jnp.sqrt(dz * dz + dy * dy + dx * dx)
    dzm = jnp.maximum(zc, 30.0 - zc)
    dym = jnp.maximum(yc, 31.0 - yc)
    dxm = jnp.maximum(xc, 31.0 - xc)
    maxd = jnp.sqrt(dzm * dzm + dym * dym + dxm * dxm)
    sd = dist / (maxd + 1e-8)
    out_ref[0] = 0.9 * sem + 0.1 * (1.0 - sd)


def _kt(ct_ref, v_ref, n_ref, s_ref):
    s_ref[...] = ct_ref[0]
    nco = lax.broadcasted_iota(jnp.int32, (NE, N), 1)

    def body(i, carry):
        s = s_ref[...]
        m = jnp.max(s, axis=1, keepdims=True)
        sel = jnp.min(jnp.where(s == m, nco, BIG), axis=1, keepdims=True)
        s_ref[...] = jnp.where(nco == sel, NEG, s)
        return m, sel

    m, sel = lax.fori_loop(0, K, body,
                           (jnp.zeros((NE, 1), jnp.float32),
                            jnp.zeros((NE, 1), jnp.int32)))
    v_ref[0] = m.reshape(1, NE)
    n_ref[0] = sel.reshape(1, NE)


def _kc(ct_ref, v_ref, n_ref, h_ref):
    j = pl.program_id(1)
    cb = ct_ref[0]
    v = v_ref[0, 0][:, None]
    nn = n_ref[0, 0][:, None]
    nrow = j * NBLK + lax.broadcasted_iota(jnp.int32, (1, NBLK), 1)
    keep = (cb > v) | ((cb == v) & (nrow <= nn))
    h_ref[0] = keep.astype(jnp.float32).T


def kernel(node_features, class_token, temperature):
    inv3, sarr = pl.pallas_call(
        _ka1,
        grid=(B, NJ),
        in_specs=[
            pl.BlockSpec((1, NBLK, C), lambda b, j: (b, j, 0)),
            pl.BlockSpec((1, 1, C), lambda b, j: (0, 0, 0)),
        ],
        out_specs=[
            pl.BlockSpec((1, NBLK, 1), lambda b, j: (b, j, 0)),
            pl.BlockSpec((1, NBLK, 1), lambda b, j: (b, j, 0)),
        ],
        out_shape=[
            jax.ShapeDtypeStruct((B, N, 1), jnp.float32),
            jax.ShapeDtypeStruct((B, N, 1), jnp.float32),
        ],
    )(node_features, class_token)

    cidx, cflat = pl.pallas_call(
        _ka2,
        grid=(B,),
        in_specs=[pl.BlockSpec((1, SUB, LANE), lambda b: (b, 0, 0))],
        out_specs=[
            pl.BlockSpec((1, 1, NE), lambda b: (b, 0, 0),
                         memory_space=pltpu.SMEM),
            pl.BlockSpec((1, 1, NE), lambda b: (b, 0, 0),
                         memory_space=pltpu.SMEM),
        ],
        out_shape=[
            jax.ShapeDtypeStruct((B, 1, NE), jnp.int32),
            jax.ShapeDtypeStruct((B, 1, NE), jnp.int32),
        ],
    )(sarr.reshape(B, SUB, LANE))

    mesh = plsc.VectorSubcoreMesh(core_axis_name="c", subcore_axis_name="s")
    centers_flat = pl.kernel(
        _kg_sc,
        mesh=mesh,
        out_type=jax.ShapeDtypeStruct((NROWS, C), jnp.float32),
        scratch_types=[
            pltpu.VMEM((RPW,), jnp.int32),
            pltpu.VMEM((RPW, C), jnp.float32),
            pltpu.SemaphoreType.DMA,
        ],
    )(node_features.reshape(B * N, C), cflat.reshape(NROWS))
    centers = centers_flat.reshape(B, NE, C)

    comb = pl.pallas_call(
        _kb,
        grid=(B, NJ),
        in_specs=[
            pl.BlockSpec((1, 1, NE), lambda b, j: (b, 0, 0)),
            pl.BlockSpec((1, NE, C), lambda b, j: (b, 0, 0)),
            pl.BlockSpec((1, NBLK, 1), lambda b, j: (b, j, 0)),
            pl.BlockSpec((1, NBLK, C), lambda b, j: (b, j, 0)),
            pl.BlockSpec(memory_space=pltpu.SMEM),
        ],
        out_specs=pl.BlockSpec((1, NE, NBLK), lambda b, j: (b, 0, j)),
        out_shape=jax.ShapeDtypeStruct((B, NE, N), jnp.float32),
    )(cidx, centers, inv3, node_features, temperature)

    v32, n32 = pl.pallas_call(
        _kt,
        grid=(B,),
        in_specs=[pl.BlockSpec((1, NE, N), lambda b: (b, 0, 0))],
        out_specs=[
            pl.BlockSpec((1, 1, NE), lambda b: (b, 0, 0)),
            pl.BlockSpec((1, 1, NE), lambda b: (b, 0, 0)),
        ],
        out_shape=[
            jax.ShapeDtypeStruct((B, 1, NE), jnp.float32),
            jax.ShapeDtypeStruct((B, 1, NE), jnp.int32),
        ],
        scratch_shapes=[pltpu.VMEM((NE, N), jnp.float32)],
    )(comb)

    H = pl.pallas_call(
        _kc,
        grid=(B, NJ),
        in_specs=[
            pl.BlockSpec((1, NE, NBLK), lambda b, j: (b, 0, j)),
            pl.BlockSpec((1, 1, NE), lambda b, j: (b, 0, 0)),
            pl.BlockSpec((1, 1, NE), lambda b, j: (b, 0, 0)),
        ],
        out_specs=pl.BlockSpec((1, NBLK, NE), lambda b, j: (b, j, 0)),
        out_shape=jax.ShapeDtypeStruct((B, N, NE), jnp.float32),
    )(comb, v32, n32)
    return H
